# baseline (speedup 1.0000x reference)
import functools

import jax
import jax.numpy as jnp
from jax import lax
from jax.experimental import pallas as pl
from jax.experimental.pallas import tpu as pltpu
from jax.experimental.pallas import tpu_sc as plsc

_LANES = 16


def kernel(features, labels, centers):
    B, D = features.shape
    n_cores, n_sub = 2, 16
    n_workers = n_cores * n_sub
    bpw = B // n_workers
    chunk = 128
    n_chunks = bpw // chunk
    d_vecs = D // _LANES

    idx2d = labels.astype(jnp.int32).reshape(B // chunk, chunk)

    mesh = plsc.VectorSubcoreMesh(core_axis_name="c", subcore_axis_name="s")

    @functools.partial(
        pl.kernel,
        mesh=mesh,
        compiler_params=pltpu.CompilerParams(use_tc_tiling_on_sc=False),
        out_type=jax.ShapeDtypeStruct((n_workers, _LANES), jnp.float32),
        scratch_types=[
            pltpu.VMEM((n_chunks, chunk), jnp.int32),
            pltpu.VMEM((bpw, D), jnp.float32),
            pltpu.VMEM((bpw, D), jnp.float32),
            pltpu.VMEM((_LANES,), jnp.float32),
            pltpu.SemaphoreType.DMA,
            pltpu.SemaphoreType.DMA,
            pltpu.SemaphoreType.DMA,
            pltpu.SemaphoreType.DMA,
            pltpu.SemaphoreType.DMA,
        ],
    )
    def run(feat_hbm, idx_hbm, cent_hbm, out_hbm,
            idx_v, feat_v, rows_v, out_v, fsem, *gsems):
        wid = lax.axis_index("s") * n_cores + lax.axis_index("c")
        base = wid * bpw

        pltpu.sync_copy(idx_hbm.at[pl.ds(wid * n_chunks, n_chunks)], idx_v)

        fcopy = pltpu.async_copy(feat_hbm.at[pl.ds(base, bpw)], feat_v, fsem)
        gcopies = []
        for j in range(n_chunks):
            gcopies.append(
                pltpu.async_copy(
                    cent_hbm.at[idx_v.at[j]],
                    rows_v.at[pl.ds(j * chunk, chunk)],
                    gsems[j],
                )
            )
        fcopy.wait()

        zero = jnp.zeros((_LANES,), jnp.float32)
        accs = (zero,) * d_vecs
        for j in range(n_chunks):
            gcopies[j].wait()

            def body(r, a, _j=j):
                row = _j * chunk + r
                out = []
                for s in range(d_vecs):
                    f = feat_v[row, pl.ds(s * _LANES, _LANES)]
                    c = rows_v[row, pl.ds(s * _LANES, _LANES)]
                    d = f - c
                    out.append(a[s] + d * d)
                return tuple(out)

            accs = lax.fori_loop(0, chunk, body, accs)

        total = (accs[0] + accs[1]) + (accs[2] + accs[3])
        out_v[...] = total
        pltpu.sync_copy(out_v, out_hbm.at[wid])

    partials = run(features, idx2d, centers)
    return jnp.sum(partials) / (B * D)


# speedup vs baseline: 1.9304x; 1.9304x over previous
import functools

import jax
import jax.numpy as jnp
from jax import lax
from jax.experimental import pallas as pl
from jax.experimental.pallas import tpu as pltpu
from jax.experimental.pallas import tpu_sc as plsc

_LANES = 16


def kernel(features, labels, centers):
    B, D = features.shape
    V = centers.shape[0]
    n_cores, n_sub = 2, 16
    n_workers = n_cores * n_sub
    cols_per_w = D // n_workers
    chunk = 4096
    n_chunks = B // chunk

    labels32 = labels.astype(jnp.int32)
    feat_t = features.T
    cent_t = centers.T

    mesh = plsc.VectorSubcoreMesh(core_axis_name="c", subcore_axis_name="s")

    @functools.partial(
        pl.kernel,
        mesh=mesh,
        compiler_params=pltpu.CompilerParams(needs_layout_passes=False),
        out_type=jax.ShapeDtypeStruct((n_workers, _LANES), jnp.float32),
        scratch_types=[
            pltpu.VMEM((V,), jnp.float32),
            pltpu.VMEM((2, chunk), jnp.int32),
            pltpu.VMEM((2, chunk), jnp.float32),
            pltpu.VMEM((_LANES,), jnp.float32),
            pltpu.SemaphoreType.DMA,
            pltpu.SemaphoreType.DMA,
            pltpu.SemaphoreType.DMA,
        ],
    )
    def run(feat_hbm, lab_hbm, cent_hbm, out_hbm,
            row_v, lab_v, fchunk_v, out_v, rsem, csem_a, csem_b):
        wid = lax.axis_index("s") * n_cores + lax.axis_index("c")
        csems = (csem_a, csem_b)

        def fire_chunk(c, k):
            sl = pl.ds(k * chunk, chunk)
            b = k % 2
            lc = pltpu.async_copy(lab_hbm.at[sl], lab_v.at[b], csems[b])
            fc = pltpu.async_copy(feat_hbm.at[c, sl], fchunk_v.at[b], csems[b])
            return lc, fc

        acc = jnp.zeros((_LANES,), jnp.float32)
        for r in range(cols_per_w):
            c = wid * cols_per_w + r
            rcopy = pltpu.async_copy(cent_hbm.at[c], row_v, rsem)
            pending = fire_chunk(c, 0)
            rcopy.wait()
            for k in range(n_chunks):
                for p in pending:
                    p.wait()
                if k + 1 < n_chunks:
                    pending = fire_chunk(c, k + 1)
                b = k % 2

                def body(i, a, _b=b):
                    sl = pl.ds(i * _LANES, _LANES)
                    idx = lab_v[_b, sl]
                    f = fchunk_v[_b, sl]
                    cv = plsc.load_gather(row_v, [idx])
                    d = f - cv
                    return a + d * d

                acc = lax.fori_loop(0, chunk // _LANES, body, acc)

        out_v[...] = acc
        pltpu.sync_copy(out_v, out_hbm.at[wid])

    partials = run(feat_t, labels32, cent_t)
    return jnp.sum(partials) / (B * D)


# speedup vs baseline: 2.3115x; 1.1974x over previous
import functools

import jax
import jax.numpy as jnp
from jax import lax
from jax.experimental import pallas as pl
from jax.experimental.pallas import tpu as pltpu
from jax.experimental.pallas import tpu_sc as plsc

_LANES = 16


def kernel(features, labels, centers):
    B, D = features.shape
    V = centers.shape[0]
    n_cores, n_sub = 2, 16
    n_workers = n_cores * n_sub
    cols_per_w = D // n_workers
    chunk = 4096
    n_chunks = B // chunk
    unroll = 4
    it_per_chunk = chunk // (_LANES * unroll)

    labels32 = labels.astype(jnp.int32)
    feat_t = features.T
    cent_t = centers.T

    mesh = plsc.VectorSubcoreMesh(core_axis_name="c", subcore_axis_name="s")

    @functools.partial(
        pl.kernel,
        mesh=mesh,
        compiler_params=pltpu.CompilerParams(needs_layout_passes=False),
        out_type=jax.ShapeDtypeStruct((n_workers, _LANES), jnp.float32),
        scratch_types=[
            pltpu.VMEM((V,), jnp.float32),
            pltpu.VMEM((B,), jnp.int32),
            pltpu.VMEM((2, chunk), jnp.float32),
            pltpu.VMEM((_LANES,), jnp.float32),
            pltpu.SemaphoreType.DMA,
            pltpu.SemaphoreType.DMA,
            pltpu.SemaphoreType.DMA,
            pltpu.SemaphoreType.DMA,
        ],
    )
    def run(feat_hbm, lab_hbm, cent_hbm, out_hbm,
            row_v, lab_v, fchunk_v, out_v, rsem, lsem, csem_a, csem_b):
        wid = lax.axis_index("s") * n_cores + lax.axis_index("c")
        csems = (csem_a, csem_b)

        def fire_chunk(c, k):
            b = k % 2
            return pltpu.async_copy(
                feat_hbm.at[c, pl.ds(k * chunk, chunk)], fchunk_v.at[b],
                csems[b])

        lcopy = pltpu.async_copy(lab_hbm, lab_v, lsem)
        rcopy = pltpu.async_copy(cent_hbm.at[wid * cols_per_w], row_v, rsem)
        lcopy.wait()

        zero = jnp.zeros((_LANES,), jnp.float32)
        accs = [zero] * unroll
        for r in range(cols_per_w):
            c = wid * cols_per_w + r
            pending = fire_chunk(c, 0)
            rcopy.wait()
            for k in range(n_chunks):
                pending.wait()
                if k + 1 < n_chunks:
                    pending = fire_chunk(c, k + 1)
                b = k % 2
                base = k * chunk

                def body(i, a, _b=b, _base=base):
                    out = []
                    for u in range(unroll):
                        off = i * (_LANES * unroll) + u * _LANES
                        idx = lab_v[pl.ds(_base + off, _LANES)]
                        f = fchunk_v[_b, pl.ds(off, _LANES)]
                        cv = plsc.load_gather(row_v, [idx])
                        d = f - cv
                        out.append(a[u] + d * d)
                    return tuple(out)

                accs = list(lax.fori_loop(0, it_per_chunk, body, tuple(accs)))

            if r + 1 < cols_per_w:
                rcopy = pltpu.async_copy(cent_hbm.at[c + 1], row_v, rsem)

        total = (accs[0] + accs[1]) + (accs[2] + accs[3])
        out_v[...] = total
        pltpu.sync_copy(out_v, out_hbm.at[wid])

    partials = run(feat_t, labels32, cent_t)
    return jnp.sum(partials) / (B * D)
